# baseline (speedup 1.0000x reference)
import jax
import jax.numpy as jnp
from jax.experimental import pallas as pl
from jax.experimental.pallas import tpu as pltpu

B, Q, D, E, K = 128, 32, 512, 300, 21
E2 = E // 2
V = 50000
CHUNKS = 8
BC = B // CHUNKS
VB = 2000


def _rbf_mus(n):
    mus = [1.0]
    if n == 1:
        return mus
    bin_size = 2.0 / (n - 1)
    mus.append(1 - bin_size / 2)
    for i in range(1, n - 1):
        mus.append(mus[i] - bin_size)
    return mus


def _rbf_neg_inv_two_sigma_sq(n):
    sigmas = [0.001] + [0.1] * (n - 1)
    return [-1.0 / (2.0 * s * s) for s in sigmas]


_MUS = _rbf_mus(K)
_NEG_C = _rbf_neg_inv_two_sigma_sq(K)


def _pack_body(t_ref, o_ref):
    u = pltpu.bitcast(t_ref[...], jnp.uint32)
    r = (u + jnp.uint32(0x7FFF) + ((u >> 16) & jnp.uint32(1))) \
        & jnp.uint32(0xFFFF0000)
    o_ref[...] = pltpu.bitcast(
        (r[:, :E2] >> 16) | (r[:, E2:] & jnp.uint32(0xFFFF0000)), jnp.int32)


def _pack_table(emb_table):
    return pl.pallas_call(
        _pack_body,
        grid=(V // VB,),
        in_specs=[pl.BlockSpec((VB, E), lambda i: (i, 0))],
        out_specs=pl.BlockSpec((VB, E2), lambda i: (i, 0)),
        out_shape=jax.ShapeDtypeStruct((V, E2), jnp.int32),
        compiler_params=pltpu.CompilerParams(
            dimension_semantics=("parallel",),
        ),
    )(emb_table)


def _unpack_bf16_pair(x):
    lo = pltpu.bitcast(jax.lax.shift_left(x, 16), jnp.float32)
    hi = pltpu.bitcast(jnp.bitwise_and(x, jnp.int32(-65536)), jnp.float32)
    return lo, hi


def _knrm_body(emb_ref, ql_ref, dl_ref, w_ref, b_ref, lps_ref, sc_ref):
    q_lo, q_hi = _unpack_bf16_pair(emb_ref[0, :Q, :])
    d_lo, d_hi = _unpack_bf16_pair(emb_ref[0, Q:, :])

    qn2 = (jnp.sum(q_lo * q_lo, axis=1, keepdims=True)
           + jnp.sum(q_hi * q_hi, axis=1, keepdims=True))
    qs = jax.lax.rsqrt(jnp.maximum(qn2, 1e-24))
    dn2 = (jnp.sum(d_lo * d_lo, axis=1, keepdims=True)
           + jnp.sum(d_hi * d_hi, axis=1, keepdims=True))
    ds = jax.lax.rsqrt(jnp.maximum(dn2, 1e-24))

    dims = (((1,), (1,)), ((), ()))
    sim = (
        jax.lax.dot_general(
            (q_lo * qs).astype(jnp.bfloat16), (d_lo * ds).astype(jnp.bfloat16),
            dims, preferred_element_type=jnp.float32)
        + jax.lax.dot_general(
            (q_hi * qs).astype(jnp.bfloat16), (d_hi * ds).astype(jnp.bfloat16),
            dims, preferred_element_type=jnp.float32)
    )

    dlen = dl_ref[0, 0, 0]
    dmask = jax.lax.broadcasted_iota(jnp.int32, (Q, D), 1) < dlen
    sim = jnp.where(dmask, sim, -30.0)

    sums = []
    for k in range(K):
        diff = sim - _MUS[k]
        p = jnp.exp(diff * diff * _NEG_C[k])
        sums.append(jnp.sum(p, axis=1, keepdims=True))
    ps = jnp.concatenate(sums, axis=1)

    lp = jnp.log(jnp.maximum(ps, 1e-10)) * 0.01

    qlen = ql_ref[0, 0, 0]
    qmask = jax.lax.broadcasted_iota(jnp.int32, (Q, K), 0) < qlen
    lpsum = jnp.sum(jnp.where(qmask, lp, 0.0), axis=0, keepdims=True)

    lps_ref[0] = lpsum

    wb = w_ref[0].astype(jnp.bfloat16).astype(jnp.float32)
    lb = lpsum.astype(jnp.bfloat16).astype(jnp.float32)
    sc_ref[0] = jnp.sum(lb * wb, axis=1, keepdims=True) + b_ref[0]


def _chunk_call(emb_c, ql_c, dl_c, w3, b3):
    return pl.pallas_call(
        _knrm_body,
        grid=(BC,),
        in_specs=[
            pl.BlockSpec((1, Q + D, E2), lambda b: (b, 0, 0)),
            pl.BlockSpec((1, 1, 1), lambda b: (b, 0, 0), memory_space=pltpu.SMEM),
            pl.BlockSpec((1, 1, 1), lambda b: (b, 0, 0), memory_space=pltpu.SMEM),
            pl.BlockSpec((1, 1, K), lambda b: (0, 0, 0)),
            pl.BlockSpec((1, 1, 1), lambda b: (0, 0, 0)),
        ],
        out_specs=(
            pl.BlockSpec((1, 1, K), lambda b: (b, 0, 0)),
            pl.BlockSpec((1, 1, 1), lambda b: (b, 0, 0)),
        ),
        out_shape=(
            jax.ShapeDtypeStruct((BC, 1, K), jnp.float32),
            jax.ShapeDtypeStruct((BC, 1, 1), jnp.float32),
        ),
        compiler_params=pltpu.CompilerParams(
            dimension_semantics=("parallel",),
        ),
    )(emb_c, ql_c, dl_c, w3, b3)


@jax.jit
def kernel(query_idx, doc_idx, query_len, doc_len, emb_table, dense_w, dense_b):
    packed = _pack_table(emb_table)
    idx = jnp.concatenate([query_idx, doc_idx], axis=1)
    ql3 = query_len.reshape(B, 1, 1)
    dl3 = doc_len.reshape(B, 1, 1)
    w3 = dense_w.reshape(1, 1, K)
    b3 = dense_b.reshape(1, 1, 1)

    lps_parts, sc_parts = [], []
    for c in range(CHUNKS):
        sl = slice(c * BC, (c + 1) * BC)
        emb_c = packed[idx[sl]]
        lps_c, sc_c = _chunk_call(emb_c, ql3[sl], dl3[sl], w3, b3)
        lps_parts.append(lps_c)
        sc_parts.append(sc_c)

    lps = jnp.concatenate(lps_parts, axis=0)
    score = jnp.concatenate(sc_parts, axis=0)
    return score[:, 0, 0], lps[:, 0, :]


# speedup vs baseline: 1.0445x; 1.0445x over previous
import jax
import jax.numpy as jnp
from jax.experimental import pallas as pl
from jax.experimental.pallas import tpu as pltpu

B, Q, D, E, K = 128, 32, 512, 300, 21
E2 = E // 2
V = 50000
CHUNKS = 2
BC = B // CHUNKS
VB = 2000


def _rbf_mus(n):
    mus = [1.0]
    if n == 1:
        return mus
    bin_size = 2.0 / (n - 1)
    mus.append(1 - bin_size / 2)
    for i in range(1, n - 1):
        mus.append(mus[i] - bin_size)
    return mus


def _rbf_neg_inv_two_sigma_sq(n):
    sigmas = [0.001] + [0.1] * (n - 1)
    return [-1.0 / (2.0 * s * s) for s in sigmas]


_MUS = _rbf_mus(K)
_NEG_C = _rbf_neg_inv_two_sigma_sq(K)


def _pack_body(t_ref, o_ref):
    u = pltpu.bitcast(t_ref[...], jnp.uint32)
    r = (u + jnp.uint32(0x7FFF) + ((u >> 16) & jnp.uint32(1))) \
        & jnp.uint32(0xFFFF0000)
    o_ref[...] = pltpu.bitcast(
        (r[:, :E2] >> 16) | (r[:, E2:] & jnp.uint32(0xFFFF0000)), jnp.int32)


def _pack_table(emb_table):
    return pl.pallas_call(
        _pack_body,
        grid=(V // VB,),
        in_specs=[pl.BlockSpec((VB, E), lambda i: (i, 0))],
        out_specs=pl.BlockSpec((VB, E2), lambda i: (i, 0)),
        out_shape=jax.ShapeDtypeStruct((V, E2), jnp.int32),
        compiler_params=pltpu.CompilerParams(
            dimension_semantics=("parallel",),
        ),
    )(emb_table)


def _unpack_bf16_pair(x):
    lo = pltpu.bitcast(jax.lax.shift_left(x, 16), jnp.float32)
    hi = pltpu.bitcast(jnp.bitwise_and(x, jnp.int32(-65536)), jnp.float32)
    return lo, hi


def _knrm_body(emb_ref, ql_ref, dl_ref, w_ref, b_ref, lps_ref, sc_ref):
    q_lo, q_hi = _unpack_bf16_pair(emb_ref[0, :Q, :])
    d_lo, d_hi = _unpack_bf16_pair(emb_ref[0, Q:, :])

    qn2 = (jnp.sum(q_lo * q_lo, axis=1, keepdims=True)
           + jnp.sum(q_hi * q_hi, axis=1, keepdims=True))
    qs = jax.lax.rsqrt(jnp.maximum(qn2, 1e-24))
    dn2 = (jnp.sum(d_lo * d_lo, axis=1, keepdims=True)
           + jnp.sum(d_hi * d_hi, axis=1, keepdims=True))
    ds = jax.lax.rsqrt(jnp.maximum(dn2, 1e-24))

    dims = (((1,), (1,)), ((), ()))
    sim = (
        jax.lax.dot_general(
            (q_lo * qs).astype(jnp.bfloat16), (d_lo * ds).astype(jnp.bfloat16),
            dims, preferred_element_type=jnp.float32)
        + jax.lax.dot_general(
            (q_hi * qs).astype(jnp.bfloat16), (d_hi * ds).astype(jnp.bfloat16),
            dims, preferred_element_type=jnp.float32)
    )

    dlen = dl_ref[0, 0, 0]
    dmask = jax.lax.broadcasted_iota(jnp.int32, (Q, D), 1) < dlen
    sim = jnp.where(dmask, sim, -30.0)

    sums = []
    for k in range(K):
        diff = sim - _MUS[k]
        p = jnp.exp(diff * diff * _NEG_C[k])
        sums.append(jnp.sum(p, axis=1, keepdims=True))
    ps = jnp.concatenate(sums, axis=1)

    lp = jnp.log(jnp.maximum(ps, 1e-10)) * 0.01

    qlen = ql_ref[0, 0, 0]
    qmask = jax.lax.broadcasted_iota(jnp.int32, (Q, K), 0) < qlen
    lpsum = jnp.sum(jnp.where(qmask, lp, 0.0), axis=0, keepdims=True)

    lps_ref[0] = lpsum

    wb = w_ref[0].astype(jnp.bfloat16).astype(jnp.float32)
    lb = lpsum.astype(jnp.bfloat16).astype(jnp.float32)
    sc_ref[0] = jnp.sum(lb * wb, axis=1, keepdims=True) + b_ref[0]


def _chunk_call(emb_c, ql_c, dl_c, w3, b3):
    return pl.pallas_call(
        _knrm_body,
        grid=(BC,),
        in_specs=[
            pl.BlockSpec((1, Q + D, E2), lambda b: (b, 0, 0)),
            pl.BlockSpec((1, 1, 1), lambda b: (b, 0, 0), memory_space=pltpu.SMEM),
            pl.BlockSpec((1, 1, 1), lambda b: (b, 0, 0), memory_space=pltpu.SMEM),
            pl.BlockSpec((1, 1, K), lambda b: (0, 0, 0)),
            pl.BlockSpec((1, 1, 1), lambda b: (0, 0, 0)),
        ],
        out_specs=(
            pl.BlockSpec((1, 1, K), lambda b: (b, 0, 0)),
            pl.BlockSpec((1, 1, 1), lambda b: (b, 0, 0)),
        ),
        out_shape=(
            jax.ShapeDtypeStruct((BC, 1, K), jnp.float32),
            jax.ShapeDtypeStruct((BC, 1, 1), jnp.float32),
        ),
        compiler_params=pltpu.CompilerParams(
            dimension_semantics=("parallel",),
        ),
    )(emb_c, ql_c, dl_c, w3, b3)


@jax.jit
def kernel(query_idx, doc_idx, query_len, doc_len, emb_table, dense_w, dense_b):
    packed = _pack_table(emb_table)
    idx = jnp.concatenate([query_idx, doc_idx], axis=1)
    ql3 = query_len.reshape(B, 1, 1)
    dl3 = doc_len.reshape(B, 1, 1)
    w3 = dense_w.reshape(1, 1, K)
    b3 = dense_b.reshape(1, 1, 1)

    lps_parts, sc_parts = [], []
    for c in range(CHUNKS):
        sl = slice(c * BC, (c + 1) * BC)
        emb_c = packed[idx[sl]]
        lps_c, sc_c = _chunk_call(emb_c, ql3[sl], dl3[sl], w3, b3)
        lps_parts.append(lps_c)
        sc_parts.append(sc_c)

    lps = jnp.concatenate(lps_parts, axis=0)
    score = jnp.concatenate(sc_parts, axis=0)
    return score[:, 0, 0], lps[:, 0, :]


# speedup vs baseline: 1.0614x; 1.0161x over previous
import jax
import jax.numpy as jnp
from jax.experimental import pallas as pl
from jax.experimental.pallas import tpu as pltpu

B, Q, D, E, K = 128, 32, 512, 300, 21
E2 = E // 2
V = 50000
CHUNKS = 4
BC = B // CHUNKS
VB = 2000


def _rbf_mus(n):
    mus = [1.0]
    if n == 1:
        return mus
    bin_size = 2.0 / (n - 1)
    mus.append(1 - bin_size / 2)
    for i in range(1, n - 1):
        mus.append(mus[i] - bin_size)
    return mus


def _rbf_neg_inv_two_sigma_sq(n):
    sigmas = [0.001] + [0.1] * (n - 1)
    return [-1.0 / (2.0 * s * s) for s in sigmas]


_MUS = _rbf_mus(K)
_NEG_C = _rbf_neg_inv_two_sigma_sq(K)


def _pack_body(t_ref, o_ref):
    x = t_ref[...]
    n2 = jnp.sum(x * x, axis=1, keepdims=True)
    xn = x / jnp.maximum(jnp.sqrt(n2), 1e-12)
    u = pltpu.bitcast(xn, jnp.uint32)
    r = (u + jnp.uint32(0x7FFF) + ((u >> 16) & jnp.uint32(1))) \
        & jnp.uint32(0xFFFF0000)
    o_ref[...] = pltpu.bitcast(
        (r[:, :E2] >> 16) | (r[:, E2:] & jnp.uint32(0xFFFF0000)), jnp.int32)


def _pack_table(emb_table):
    return pl.pallas_call(
        _pack_body,
        grid=(V // VB,),
        in_specs=[pl.BlockSpec((VB, E), lambda i: (i, 0))],
        out_specs=pl.BlockSpec((VB, E2), lambda i: (i, 0)),
        out_shape=jax.ShapeDtypeStruct((V, E2), jnp.int32),
        compiler_params=pltpu.CompilerParams(
            dimension_semantics=("parallel",),
        ),
    )(emb_table)


def _unpack_bf16_pair(x):
    lo = pltpu.bitcast(jax.lax.shift_left(x, 16), jnp.float32)
    hi = pltpu.bitcast(jnp.bitwise_and(x, jnp.int32(-65536)), jnp.float32)
    return lo, hi


def _knrm_body(emb_ref, ql_ref, dl_ref, w_ref, b_ref, lps_ref, sc_ref):
    q_lo, q_hi = _unpack_bf16_pair(emb_ref[0, :Q, :])
    d_lo, d_hi = _unpack_bf16_pair(emb_ref[0, Q:, :])

    dims = (((1,), (1,)), ((), ()))
    sim = (
        jax.lax.dot_general(
            q_lo.astype(jnp.bfloat16), d_lo.astype(jnp.bfloat16),
            dims, preferred_element_type=jnp.float32)
        + jax.lax.dot_general(
            q_hi.astype(jnp.bfloat16), d_hi.astype(jnp.bfloat16),
            dims, preferred_element_type=jnp.float32)
    )

    dlen = dl_ref[0, 0, 0]
    dmask = jax.lax.broadcasted_iota(jnp.int32, (Q, D), 1) < dlen
    sim = jnp.where(dmask, sim, -30.0)

    sums = []
    for k in range(K):
        diff = sim - _MUS[k]
        p = jnp.exp(diff * diff * _NEG_C[k])
        sums.append(jnp.sum(p, axis=1, keepdims=True))
    ps = jnp.concatenate(sums, axis=1)

    lp = jnp.log(jnp.maximum(ps, 1e-10)) * 0.01

    qlen = ql_ref[0, 0, 0]
    qmask = jax.lax.broadcasted_iota(jnp.int32, (Q, K), 0) < qlen
    lpsum = jnp.sum(jnp.where(qmask, lp, 0.0), axis=0, keepdims=True)

    lps_ref[0] = lpsum

    wb = w_ref[0].astype(jnp.bfloat16).astype(jnp.float32)
    lb = lpsum.astype(jnp.bfloat16).astype(jnp.float32)
    sc_ref[0] = jnp.sum(lb * wb, axis=1, keepdims=True) + b_ref[0]


def _chunk_call(emb_c, ql_c, dl_c, w3, b3):
    return pl.pallas_call(
        _knrm_body,
        grid=(BC,),
        in_specs=[
            pl.BlockSpec((1, Q + D, E2), lambda b: (b, 0, 0)),
            pl.BlockSpec((1, 1, 1), lambda b: (b, 0, 0), memory_space=pltpu.SMEM),
            pl.BlockSpec((1, 1, 1), lambda b: (b, 0, 0), memory_space=pltpu.SMEM),
            pl.BlockSpec((1, 1, K), lambda b: (0, 0, 0)),
            pl.BlockSpec((1, 1, 1), lambda b: (0, 0, 0)),
        ],
        out_specs=(
            pl.BlockSpec((1, 1, K), lambda b: (b, 0, 0)),
            pl.BlockSpec((1, 1, 1), lambda b: (b, 0, 0)),
        ),
        out_shape=(
            jax.ShapeDtypeStruct((BC, 1, K), jnp.float32),
            jax.ShapeDtypeStruct((BC, 1, 1), jnp.float32),
        ),
        compiler_params=pltpu.CompilerParams(
            dimension_semantics=("parallel",),
        ),
    )(emb_c, ql_c, dl_c, w3, b3)


@jax.jit
def kernel(query_idx, doc_idx, query_len, doc_len, emb_table, dense_w, dense_b):
    packed = _pack_table(emb_table)
    idx = jnp.concatenate([query_idx, doc_idx], axis=1)
    ql3 = query_len.reshape(B, 1, 1)
    dl3 = doc_len.reshape(B, 1, 1)
    w3 = dense_w.reshape(1, 1, K)
    b3 = dense_b.reshape(1, 1, 1)

    lps_parts, sc_parts = [], []
    for c in range(CHUNKS):
        sl = slice(c * BC, (c + 1) * BC)
        emb_c = packed[idx[sl]]
        lps_c, sc_c = _chunk_call(emb_c, ql3[sl], dl3[sl], w3, b3)
        lps_parts.append(lps_c)
        sc_parts.append(sc_c)

    lps = jnp.concatenate(lps_parts, axis=0)
    score = jnp.concatenate(sc_parts, axis=0)
    return score[:, 0, 0], lps[:, 0, :]
